# baseline (speedup 1.0000x reference)
import functools

import jax
import jax.numpy as jnp
from jax import lax
from jax.experimental import pallas as pl
from jax.experimental.pallas import tpu as pltpu
from jax.experimental.pallas import tpu_sc as plsc

_B = 16384
_C = 1000


_TCOLS = 12288
_BN = 1024


def _tc_body(q_ref, t_ref, out_ref):
    i = pl.program_id(0)

    @pl.when(i == 0)
    def _init():
        out_ref[...] = jnp.zeros((1, 1), jnp.float32)

    x = q_ref[...]
    tcol = t_ref[...].reshape(1, _BN)
    rows = lax.broadcasted_iota(jnp.int32, (_C, _BN), 0)
    picked = jnp.where(rows == tcol, x, 0.0)
    out_ref[...] += (jnp.sum(x * x) - 2.0 * jnp.sum(picked)).reshape(1, 1)


def _tc_part(q, t3):
    return pl.pallas_call(
        _tc_body,
        grid=(_TCOLS // _BN,),
        in_specs=[
            pl.BlockSpec((_C, _BN), lambda i: (0, i)),
            pl.BlockSpec((1, 1, _BN), lambda i: (i, 0, 0)),
        ],
        out_specs=pl.BlockSpec((1, 1), lambda i: (0, 0)),
        out_shape=jax.ShapeDtypeStruct((1, 1), jnp.float32),
    )(q, t3)


_NC = 2
_NS = 16
_NW = _NC * _NS
_SCOLS = _B - _TCOLS
_CPW = _SCOLS // _NW
_CR = 40
_NBUF = 5
_NCH = _C // _CR
_NG = _CPW // 16

_mesh = plsc.VectorSubcoreMesh(core_axis_name="c", subcore_axis_name="s")


@functools.partial(
    pl.kernel,
    mesh=_mesh,
    compiler_params=pltpu.CompilerParams(use_tc_tiling_on_sc=True),
    out_type=jax.ShapeDtypeStruct((_NW, 16), jnp.float32),
    scratch_types=[
        pltpu.VMEM((_CR, _CPW), jnp.float32),
        pltpu.VMEM((_CR, _CPW), jnp.float32),
        pltpu.VMEM((_CR, _CPW), jnp.float32),
        pltpu.VMEM((_CR, _CPW), jnp.float32),
        pltpu.VMEM((_CR, _CPW), jnp.float32),
        pltpu.VMEM((_CPW,), jnp.int32),
        pltpu.VMEM((16,), jnp.float32),
        pltpu.SemaphoreType.DMA,
        pltpu.SemaphoreType.DMA,
        pltpu.SemaphoreType.DMA,
        pltpu.SemaphoreType.DMA,
        pltpu.SemaphoreType.DMA,
    ],
)
def _brier_sc(q_hbm, t_hbm, out_hbm, buf0, buf1, buf2, buf3, buf4,
              t_v, acc_v, sem0, sem1, sem2, sem3, sem4):
    wid = lax.axis_index("s") * _NC + lax.axis_index("c")
    cbase = _TCOLS + wid * _CPW
    pltpu.sync_copy(t_hbm.at[pl.ds(cbase, _CPW)], t_v)
    bufs = (buf0, buf1, buf2, buf3, buf4)
    sems = (sem0, sem1, sem2, sem3, sem4)

    def start(c, b):
        pltpu.async_copy(
            q_hbm.at[pl.ds(c * _CR, _CR), pl.ds(cbase, _CPW)],
            bufs[b], sems[b])

    def drain(c, b):
        pltpu.make_async_copy(
            q_hbm.at[pl.ds(c * _CR, _CR), pl.ds(cbase, _CPW)],
            bufs[b], sems[b]).wait()

    for _b in range(_NBUF):
        start(_b, _b)

    tvs = tuple(t_v[pl.ds(g * 16, 16)] for g in range(_NG))
    zero16 = jnp.zeros((16,), jnp.float32)
    init = (zero16, zero16, zero16, zero16)

    def chunk_step(c5, accs):
        for b in range(_NBUF):
            c = c5 * _NBUF + b
            buf = bufs[b]
            drain(c, b)
            r0 = c * _CR

            def row_body(rr, accs):
                s0, s1, p0, p1 = accs
                rg = r0 + rr
                for g in range(_NG):
                    v = buf[rr, pl.ds(g * 16, 16)]
                    sel = jnp.where(tvs[g] == rg, v, 0.0)
                    if g % 2 == 0:
                        s0 = s0 + v * v
                        p0 = p0 + sel
                    else:
                        s1 = s1 + v * v
                        p1 = p1 + sel
                return (s0, s1, p0, p1)

            accs = lax.fori_loop(0, _CR, row_body, accs)

            @pl.when(c + _NBUF < _NCH)
            def _refill():
                start(c + _NBUF, b)

        return accs

    s0, s1, p0, p1 = lax.fori_loop(0, _NCH // _NBUF, chunk_step, init)
    acc_v[...] = (s0 + s1) - 2.0 * (p0 + p1)
    pltpu.sync_copy(acc_v, out_hbm.at[wid])


def kernel(p, t):
    t32 = t.astype(jnp.int32)
    q = p.T
    partials = _brier_sc(q, t32)
    tc_sum = _tc_part(q, t32.reshape(_B // _BN, 1, _BN))[0, 0]
    return (tc_sum + jnp.sum(partials)) / _B + 1.0


# speedup vs baseline: 1.0186x; 1.0186x over previous
import functools

import jax
import jax.numpy as jnp
from jax import lax
from jax.experimental import pallas as pl
from jax.experimental.pallas import tpu as pltpu
from jax.experimental.pallas import tpu_sc as plsc

_B = 16384
_C = 1000


_TCOLS = 12288
_BN = 2048


def _tc_body(q_ref, t_ref, out_ref):
    i = pl.program_id(0)

    @pl.when(i == 0)
    def _init():
        out_ref[...] = jnp.zeros((1, 1), jnp.float32)

    x = q_ref[...]
    tcol = t_ref[...].reshape(1, _BN)
    rows = lax.broadcasted_iota(jnp.int32, (_C, _BN), 0)
    picked = jnp.where(rows == tcol, x, 0.0)
    out_ref[...] += (jnp.sum(x * x) - 2.0 * jnp.sum(picked)).reshape(1, 1)


def _tc_part(q, t3):
    return pl.pallas_call(
        _tc_body,
        grid=(_TCOLS // _BN,),
        in_specs=[
            pl.BlockSpec((_C, _BN), lambda i: (0, i)),
            pl.BlockSpec((1, 1, _BN), lambda i: (i, 0, 0)),
        ],
        out_specs=pl.BlockSpec((1, 1), lambda i: (0, 0)),
        out_shape=jax.ShapeDtypeStruct((1, 1), jnp.float32),
    )(q, t3)


_NC = 2
_NS = 16
_NW = _NC * _NS
_SCOLS = _B - _TCOLS
_CPW = _SCOLS // _NW
_CR = 40
_NBUF = 5
_NCH = _C // _CR
_NG = _CPW // 16

_mesh = plsc.VectorSubcoreMesh(core_axis_name="c", subcore_axis_name="s")


@functools.partial(
    pl.kernel,
    mesh=_mesh,
    compiler_params=pltpu.CompilerParams(use_tc_tiling_on_sc=True),
    out_type=jax.ShapeDtypeStruct((_NW, 16), jnp.float32),
    scratch_types=[
        pltpu.VMEM((_CR, _CPW), jnp.float32),
        pltpu.VMEM((_CR, _CPW), jnp.float32),
        pltpu.VMEM((_CR, _CPW), jnp.float32),
        pltpu.VMEM((_CR, _CPW), jnp.float32),
        pltpu.VMEM((_CR, _CPW), jnp.float32),
        pltpu.VMEM((_CPW,), jnp.int32),
        pltpu.VMEM((16,), jnp.float32),
        pltpu.SemaphoreType.DMA,
        pltpu.SemaphoreType.DMA,
        pltpu.SemaphoreType.DMA,
        pltpu.SemaphoreType.DMA,
        pltpu.SemaphoreType.DMA,
    ],
)
def _brier_sc(q_hbm, t_hbm, out_hbm, buf0, buf1, buf2, buf3, buf4,
              t_v, acc_v, sem0, sem1, sem2, sem3, sem4):
    wid = lax.axis_index("s") * _NC + lax.axis_index("c")
    cbase = _TCOLS + wid * _CPW
    pltpu.sync_copy(t_hbm.at[pl.ds(cbase, _CPW)], t_v)
    bufs = (buf0, buf1, buf2, buf3, buf4)
    sems = (sem0, sem1, sem2, sem3, sem4)

    def start(c, b):
        pltpu.async_copy(
            q_hbm.at[pl.ds(c * _CR, _CR), pl.ds(cbase, _CPW)],
            bufs[b], sems[b])

    def drain(c, b):
        pltpu.make_async_copy(
            q_hbm.at[pl.ds(c * _CR, _CR), pl.ds(cbase, _CPW)],
            bufs[b], sems[b]).wait()

    for _b in range(_NBUF):
        start(_b, _b)

    tvs = tuple(t_v[pl.ds(g * 16, 16)] for g in range(_NG))
    zero16 = jnp.zeros((16,), jnp.float32)
    init = (zero16, zero16, zero16, zero16)

    def chunk_step(c5, accs):
        for b in range(_NBUF):
            c = c5 * _NBUF + b
            buf = bufs[b]
            drain(c, b)
            r0 = c * _CR

            def row_body(rr, accs):
                s0, s1, p0, p1 = accs
                rg = r0 + rr
                for g in range(_NG):
                    v = buf[rr, pl.ds(g * 16, 16)]
                    sel = jnp.where(tvs[g] == rg, v, 0.0)
                    if g % 2 == 0:
                        s0 = s0 + v * v
                        p0 = p0 + sel
                    else:
                        s1 = s1 + v * v
                        p1 = p1 + sel
                return (s0, s1, p0, p1)

            accs = lax.fori_loop(0, _CR, row_body, accs)

            @pl.when(c + _NBUF < _NCH)
            def _refill():
                start(c + _NBUF, b)

        return accs

    s0, s1, p0, p1 = lax.fori_loop(0, _NCH // _NBUF, chunk_step, init)
    acc_v[...] = (s0 + s1) - 2.0 * (p0 + p1)
    pltpu.sync_copy(acc_v, out_hbm.at[wid])


def kernel(p, t):
    t32 = t.astype(jnp.int32)
    q = p.T
    partials = _brier_sc(q, t32)
    tc_sum = _tc_part(q, t32.reshape(_B // _BN, 1, _BN))[0, 0]
    return (tc_sum + jnp.sum(partials)) / _B + 1.0
